# Initial kernel scaffold; baseline (speedup 1.0000x reference)
import jax
import jax.numpy as jnp
from jax.experimental import pallas as pl


def kernel(x, edge_index, W1, b1, W2, b2):
    raise NotImplementedError("write your pallas kernel here")


# speedup vs baseline: 14.7321x; 14.7321x over previous
import functools

import jax
import jax.numpy as jnp
from jax import lax
from jax.experimental import pallas as pl
from jax.experimental.pallas import tpu as pltpu
from jax.experimental.pallas import tpu_sc as plsc

N = 10000
D = 128
E = 320000

NC = 2
NS = 16
NW = NC * NS
PER_W = E // NW
CHUNK = 80
NCHUNK = PER_W // CHUNK
N_PAD = 10240
ROWS_PER_TILE = N_PAD // NS
ZR = 128
DEG_CHUNK = 2000

_mesh = plsc.VectorSubcoreMesh(
    core_axis_name="c", subcore_axis_name="s", num_cores=NC, num_subcores=NS
)
_sc_params = pltpu.CompilerParams(needs_layout_passes=False)


@functools.partial(
    pl.kernel,
    out_type=jax.ShapeDtypeStruct((NW * N,), jnp.float32),
    mesh=_mesh,
    scratch_types=[
        pltpu.VMEM((N,), jnp.float32),
        pltpu.VMEM((DEG_CHUNK,), jnp.int32),
    ],
    compiler_params=_sc_params,
)
def _deg_kernel(dst_hbm, out_hbm, hist, idxv):
    cid = lax.axis_index("c")
    sid = lax.axis_index("s")
    wid = sid * NC + cid
    zeros16 = jnp.zeros((16,), jnp.float32)
    ones16 = jnp.ones((16,), jnp.float32)

    def zero_body(i, carry):
        hist[pl.ds(i * 16, 16)] = zeros16
        return carry

    lax.fori_loop(0, N // 16, zero_body, 0)

    def chunk_body(c, carry):
        pltpu.sync_copy(
            dst_hbm.at[pl.ds(wid * PER_W + c * DEG_CHUNK, DEG_CHUNK)], idxv
        )

        def inner(j, carry2):
            idx = idxv[pl.ds(j * 16, 16)]
            plsc.addupdate_scatter(hist, [idx], ones16)
            return carry2

        lax.fori_loop(0, DEG_CHUNK // 16, inner, 0)
        return carry

    lax.fori_loop(0, PER_W // DEG_CHUNK, chunk_body, 0)
    pltpu.sync_copy(hist, out_hbm.at[pl.ds(wid * N, N)])


@functools.partial(
    pl.kernel,
    out_type=jax.ShapeDtypeStruct((NC, N_PAD, D), jnp.float32),
    mesh=_mesh,
    scratch_types=[
        pltpu.VMEM((CHUNK,), jnp.int32),
        pltpu.VMEM((CHUNK,), jnp.int32),
        pltpu.VMEM((CHUNK, D), jnp.float32),
        pltpu.VMEM((ZR, D), jnp.float32),
        pltpu.VMEM_SHARED((N_PAD, D), jnp.float32),
        pltpu.SemaphoreType.DMA,
    ],
    compiler_params=_sc_params,
)
def _edge_kernel(g_hbm, src_hbm, dst_hbm, out_hbm, srcv, dstv, rowsv, zbuf, acc, sem):
    cid = lax.axis_index("c")
    sid = lax.axis_index("s")
    wid = sid * NC + cid
    base_row = sid * ROWS_PER_TILE
    zeros16 = jnp.zeros((16,), jnp.float32)

    def zrow(i, carry):
        def zcol(j, carry2):
            zbuf[i, pl.ds(j * 16, 16)] = zeros16
            return carry2

        lax.fori_loop(0, D // 16, zcol, 0)
        return carry

    lax.fori_loop(0, ZR, zrow, 0)

    def init_body(k, carry):
        pltpu.sync_copy(zbuf, acc.at[pl.ds(base_row + k * ZR, ZR)])
        return carry

    lax.fori_loop(0, ROWS_PER_TILE // ZR, init_body, 0)
    plsc.subcore_barrier()

    def chunk_body(c, carry):
        base = wid * PER_W + c * CHUNK
        pltpu.sync_copy(src_hbm.at[pl.ds(base, CHUNK)], srcv)
        pltpu.async_copy(g_hbm.at[srcv], rowsv, sem).wait()
        pltpu.sync_copy(dst_hbm.at[pl.ds(base, CHUNK)], dstv)
        pltpu.sync_copy(rowsv, acc.at[dstv], add=True)
        return carry

    lax.fori_loop(0, NCHUNK, chunk_body, 0)
    plsc.subcore_barrier()

    def out_body(k, carry):
        r = base_row + k * ZR
        pltpu.sync_copy(acc.at[pl.ds(r, ZR)], zbuf)
        pltpu.sync_copy(zbuf, out_hbm.at[cid, pl.ds(r, ZR)])
        return carry

    lax.fori_loop(0, ROWS_PER_TILE // ZR, out_body, 0)


R = 1024


def _dinv_from_hist(hist_blk):
    deg = jnp.sum(hist_blk, axis=0) + 1.0
    return lax.rsqrt(deg)


def _pre_body(hist_ref, x_ref, w1_ref, g1_ref):
    dinv = _dinv_from_hist(hist_ref[...])[:, None]
    h = jnp.dot(x_ref[...], w1_ref[...], preferred_element_type=jnp.float32)
    g1_ref[...] = h * dinv


def _mid_body(p_ref, g1_ref, hist_ref, b1_ref, w2_ref, g2_ref):
    dinv = _dinv_from_hist(hist_ref[...])[:, None]
    h1 = jnp.maximum(dinv * (p_ref[0] + p_ref[1] + g1_ref[...]) + b1_ref[...], 0.0)
    g2_ref[...] = (
        jnp.dot(h1, w2_ref[...], preferred_element_type=jnp.float32) * dinv
    )


def _post_body(q_ref, g2_ref, hist_ref, b2_ref, out_ref):
    dinv = _dinv_from_hist(hist_ref[...])[:, None]
    out_ref[...] = jnp.maximum(
        dinv * (q_ref[0] + q_ref[1] + g2_ref[...]) + b2_ref[...], 0.0
    )


_row_spec = pl.BlockSpec((R, D), lambda i: (i, 0))
_pair_spec = pl.BlockSpec((NC, R, D), lambda i: (0, i, 0))
_hist_spec = pl.BlockSpec((NW, R), lambda i: (0, i))
_w_spec = pl.BlockSpec((D, D), lambda i: (0, 0))
_b_spec = pl.BlockSpec((D,), lambda i: (0,))
_nd_shape = jax.ShapeDtypeStruct((N, D), jnp.float32)
_grid = (pl.cdiv(N, R),)

_pre_call = pl.pallas_call(
    _pre_body,
    grid=_grid,
    in_specs=[_hist_spec, _row_spec, _w_spec],
    out_specs=_row_spec,
    out_shape=_nd_shape,
)

_mid_call = pl.pallas_call(
    _mid_body,
    grid=_grid,
    in_specs=[_pair_spec, _row_spec, _hist_spec, _b_spec, _w_spec],
    out_specs=_row_spec,
    out_shape=_nd_shape,
)

_post_call = pl.pallas_call(
    _post_body,
    grid=_grid,
    in_specs=[_pair_spec, _row_spec, _hist_spec, _b_spec],
    out_specs=_row_spec,
    out_shape=_nd_shape,
)


def kernel(x, edge_index, W1, b1, W2, b2):
    src = edge_index[0]
    dst = edge_index[1]
    hist = jnp.reshape(_deg_kernel(dst), (NW, N))
    g1 = _pre_call(hist, x, W1)
    p = _edge_kernel(g1, src, dst)
    g2 = _mid_call(p, g1, hist, b1, W2)
    q = _edge_kernel(g2, src, dst)
    return _post_call(q, g2, hist, b2)
